# Initial kernel scaffold; baseline (speedup 1.0000x reference)
import jax
import jax.numpy as jnp
from jax.experimental import pallas as pl


def kernel(z, u, w, b):
    raise NotImplementedError("write your pallas kernel here")


# speedup vs baseline: 7.2014x; 7.2014x over previous
import jax
import jax.numpy as jnp
from jax.experimental import pallas as pl
from jax.experimental.pallas import tpu as pltpu

_CH = 80_000
_NT = 16


def _stats_kernel(z_ref, u_ref, w_ref, a_ref, c_ref):
    k = pl.program_id(1)

    @pl.when(k == 0)
    def _():
        a_ref[...] = jnp.zeros_like(a_ref)
        c_ref[...] = jnp.zeros_like(c_ref)

    w_blk = w_ref[...]
    u_blk = u_ref[...]
    z_blk = z_ref[...]
    a = jax.lax.dot_general(
        w_blk, u_blk, (((1,), (1,)), ((), ())),
        preferred_element_type=jnp.float32)
    a_ref[...] += a[None]
    c_ref[...] += jnp.sum(w_blk * z_blk, axis=1, keepdims=True)[None]


def _apply_kernel(a_ref, c_ref, b_ref, z_ref, u_ref, zo_ref, ld_ref, h_s):
    k = pl.program_id(1)

    @pl.when(k == 0)
    def _():
        A = a_ref[0] + a_ref[1]
        base = c_ref[0] + c_ref[1] + b_ref[...]
        acc = jnp.zeros((_NT, 1), jnp.float32)
        hcol = jnp.zeros((_NT, 1), jnp.float32)
        ld = jnp.zeros((1, 1), jnp.float32)
        row = jax.lax.broadcasted_iota(jnp.int32, (_NT, 1), 0)
        for t in range(_NT):
            inner = base[t:t + 1, :] + acc[t:t + 1, :]
            h = jnp.tanh(inner)
            d = A[t:t + 1, t:t + 1]
            ld = ld + jnp.log(jnp.abs(1.0 + (1.0 - h * h) * d))
            acc = acc + A[:, t:t + 1] * h
            hcol = hcol + jnp.where(row == t, h, 0.0)
        h_s[...] = hcol
        ld_ref[...] = ld[None]

    zo_ref[...] = z_ref[...] + jnp.sum(
        u_ref[...] * h_s[...], axis=0, keepdims=True)


def kernel(z, u, w, b):
    dim = z.shape[0]
    nk = dim // _CH
    k1 = nk // 2
    z2 = z.reshape(1, dim)

    a_parts, c_parts = pl.pallas_call(
        _stats_kernel,
        grid=(2, k1),
        in_specs=[
            pl.BlockSpec((1, _CH), lambda i, k: (0, i * k1 + k)),
            pl.BlockSpec((_NT, _CH), lambda i, k: (0, i * k1 + k)),
            pl.BlockSpec((_NT, _CH), lambda i, k: (0, i * k1 + k)),
        ],
        out_specs=[
            pl.BlockSpec((1, _NT, _NT), lambda i, k: (i, 0, 0)),
            pl.BlockSpec((1, _NT, 1), lambda i, k: (i, 0, 0)),
        ],
        out_shape=[
            jax.ShapeDtypeStruct((2, _NT, _NT), jnp.float32),
            jax.ShapeDtypeStruct((2, _NT, 1), jnp.float32),
        ],
        compiler_params=pltpu.CompilerParams(
            dimension_semantics=("parallel", "arbitrary")),
        name="flow_stats",
    )(z2, u, w)

    z_out2, ld = pl.pallas_call(
        _apply_kernel,
        grid=(2, k1),
        in_specs=[
            pl.BlockSpec((2, _NT, _NT), lambda i, k: (0, 0, 0)),
            pl.BlockSpec((2, _NT, 1), lambda i, k: (0, 0, 0)),
            pl.BlockSpec((_NT, 1), lambda i, k: (0, 0)),
            pl.BlockSpec((1, _CH), lambda i, k: (0, i * k1 + k)),
            pl.BlockSpec((_NT, _CH), lambda i, k: (0, i * k1 + k)),
        ],
        out_specs=[
            pl.BlockSpec((1, _CH), lambda i, k: (0, i * k1 + k)),
            pl.BlockSpec((1, 1, 1), lambda i, k: (i, 0, 0)),
        ],
        out_shape=[
            jax.ShapeDtypeStruct((1, dim), jnp.float32),
            jax.ShapeDtypeStruct((2, 1, 1), jnp.float32),
        ],
        scratch_shapes=[pltpu.VMEM((_NT, 1), jnp.float32)],
        compiler_params=pltpu.CompilerParams(
            dimension_semantics=("parallel", "arbitrary")),
        name="flow_apply",
    )(a_parts, c_parts, b, z2, u)

    return z_out2.reshape(dim), ld[0, 0, 0]


# speedup vs baseline: 7.4353x; 1.0325x over previous
import jax
import jax.numpy as jnp
from jax.experimental import pallas as pl
from jax.experimental.pallas import tpu as pltpu

_CH = 80_000
_NT = 16


def _stats_kernel(z_ref, u_ref, w_ref, a_ref, c_ref):
    k = pl.program_id(1)

    @pl.when(k == 0)
    def _():
        a_ref[...] = jnp.zeros_like(a_ref)
        c_ref[...] = jnp.zeros_like(c_ref)

    w_blk = w_ref[...]
    u_blk = u_ref[...]
    z_blk = z_ref[...]
    a = jax.lax.dot_general(
        w_blk, u_blk, (((1,), (1,)), ((), ())),
        preferred_element_type=jnp.float32)
    a_ref[...] += a[None]
    c_ref[...] += jnp.sum(w_blk * z_blk, axis=1, keepdims=True)[None]


def _apply_kernel(a_ref, c_ref, b_ref, z_ref, u_ref, zo_ref, ld_ref, h_s):
    k = pl.program_id(1)

    @pl.when(k == 0)
    def _():
        A = a_ref[0] + a_ref[1]
        base = c_ref[0] + c_ref[1] + b_ref[...]
        acc = jnp.zeros((_NT, 1), jnp.float32)
        hrow = jnp.zeros((1, _NT), jnp.float32)
        ld = jnp.zeros((1, 1), jnp.float32)
        lane = jax.lax.broadcasted_iota(jnp.int32, (1, _NT), 1)
        for t in range(_NT):
            inner = base[t:t + 1, :] + acc[t:t + 1, :]
            h = jnp.tanh(inner)
            d = A[t:t + 1, t:t + 1]
            ld = ld + jnp.log(jnp.abs(1.0 + (1.0 - h * h) * d))
            acc = acc + A[:, t:t + 1] * h
            hrow = hrow + jnp.where(lane == t, h, 0.0)
        h_s[...] = hrow
        ld_ref[...] = ld[None]

    zo_ref[...] = z_ref[...] + jax.lax.dot_general(
        h_s[...], u_ref[...], (((1,), (0,)), ((), ())),
        preferred_element_type=jnp.float32)


def kernel(z, u, w, b):
    dim = z.shape[0]
    nk = dim // _CH
    k1 = nk // 2
    z2 = z.reshape(1, dim)

    a_parts, c_parts = pl.pallas_call(
        _stats_kernel,
        grid=(2, k1),
        in_specs=[
            pl.BlockSpec((1, _CH), lambda i, k: (0, i * k1 + k)),
            pl.BlockSpec((_NT, _CH), lambda i, k: (0, i * k1 + k)),
            pl.BlockSpec((_NT, _CH), lambda i, k: (0, i * k1 + k)),
        ],
        out_specs=[
            pl.BlockSpec((1, _NT, _NT), lambda i, k: (i, 0, 0)),
            pl.BlockSpec((1, _NT, 1), lambda i, k: (i, 0, 0)),
        ],
        out_shape=[
            jax.ShapeDtypeStruct((2, _NT, _NT), jnp.float32),
            jax.ShapeDtypeStruct((2, _NT, 1), jnp.float32),
        ],
        compiler_params=pltpu.CompilerParams(
            dimension_semantics=("parallel", "arbitrary")),
        name="flow_stats",
    )(z2, u, w)

    z_out2, ld = pl.pallas_call(
        _apply_kernel,
        grid=(2, k1),
        in_specs=[
            pl.BlockSpec((2, _NT, _NT), lambda i, k: (0, 0, 0)),
            pl.BlockSpec((2, _NT, 1), lambda i, k: (0, 0, 0)),
            pl.BlockSpec((_NT, 1), lambda i, k: (0, 0)),
            pl.BlockSpec((1, _CH), lambda i, k: (0, i * k1 + k)),
            pl.BlockSpec((_NT, _CH), lambda i, k: (0, i * k1 + k)),
        ],
        out_specs=[
            pl.BlockSpec((1, _CH), lambda i, k: (0, i * k1 + k)),
            pl.BlockSpec((1, 1, 1), lambda i, k: (i, 0, 0)),
        ],
        out_shape=[
            jax.ShapeDtypeStruct((1, dim), jnp.float32),
            jax.ShapeDtypeStruct((2, 1, 1), jnp.float32),
        ],
        scratch_shapes=[pltpu.VMEM((1, _NT), jnp.float32)],
        compiler_params=pltpu.CompilerParams(
            dimension_semantics=("parallel", "arbitrary")),
        name="flow_apply",
    )(a_parts, c_parts, b, z2, u)

    return z_out2.reshape(dim), ld[0, 0, 0]


# speedup vs baseline: 7.4950x; 1.0080x over previous
import jax
import jax.numpy as jnp
from jax.experimental import pallas as pl
from jax.experimental.pallas import tpu as pltpu

_CH = 160_000
_NT = 16


def _flow_kernel(b_ref, z_ref, u_ref, w_ref, zo_ref, ld_ref,
                 a_s, c_s, h_s):
    k = pl.program_id(0)
    nk = pl.num_programs(0) // 2

    @pl.when(k == 0)
    def _():
        a_s[...] = jnp.zeros_like(a_s)
        c_s[...] = jnp.zeros_like(c_s)

    @pl.when(k < nk)
    def _():
        w_blk = w_ref[...]
        u_blk = u_ref[...]
        z_blk = z_ref[...]
        a_s[...] += jax.lax.dot_general(
            w_blk, u_blk, (((1,), (1,)), ((), ())),
            preferred_element_type=jnp.float32)
        c_s[...] += jnp.sum(w_blk * z_blk, axis=1, keepdims=True)

    @pl.when(k == nk)
    def _():
        A = a_s[...]
        base = c_s[...] + b_ref[...]
        acc = jnp.zeros((_NT, 1), jnp.float32)
        hrow = jnp.zeros((1, _NT), jnp.float32)
        ld = jnp.zeros((1, 1), jnp.float32)
        lane = jax.lax.broadcasted_iota(jnp.int32, (1, _NT), 1)
        for t in range(_NT):
            inner = base[t:t + 1, :] + acc[t:t + 1, :]
            h = jnp.tanh(inner)
            d = A[t:t + 1, t:t + 1]
            ld = ld + jnp.log(jnp.abs(1.0 + (1.0 - h * h) * d))
            acc = acc + A[:, t:t + 1] * h
            hrow = hrow + jnp.where(lane == t, h, 0.0)
        h_s[...] = hrow
        ld_ref[...] = ld

    @pl.when(k >= nk)
    def _():
        zo_ref[...] = z_ref[...] + jax.lax.dot_general(
            h_s[...], u_ref[...], (((1,), (0,)), ((), ())),
            preferred_element_type=jnp.float32)


def kernel(z, u, w, b):
    dim = z.shape[0]
    nk = dim // _CH
    z2 = z.reshape(1, dim)

    z_out2, ld = pl.pallas_call(
        _flow_kernel,
        grid=(2 * nk,),
        in_specs=[
            pl.BlockSpec((_NT, 1), lambda k: (0, 0)),
            pl.BlockSpec((1, _CH), lambda k: (0, jax.lax.rem(k, nk))),
            pl.BlockSpec((_NT, _CH), lambda k: (0, jax.lax.rem(k, nk))),
            pl.BlockSpec((_NT, _CH), lambda k: (0, jnp.minimum(k, nk - 1))),
        ],
        out_specs=[
            pl.BlockSpec((1, _CH), lambda k: (0, jnp.maximum(k - nk, 0))),
            pl.BlockSpec((1, 1), lambda k: (0, 0)),
        ],
        out_shape=[
            jax.ShapeDtypeStruct((1, dim), jnp.float32),
            jax.ShapeDtypeStruct((1, 1), jnp.float32),
        ],
        scratch_shapes=[
            pltpu.VMEM((_NT, _NT), jnp.float32),
            pltpu.VMEM((_NT, 1), jnp.float32),
            pltpu.VMEM((1, _NT), jnp.float32),
        ],
        compiler_params=pltpu.CompilerParams(
            dimension_semantics=("arbitrary",),
            vmem_limit_bytes=56 * 1024 * 1024),
        name="flow_fused",
    )(b, z2, u, w)

    return z_out2.reshape(dim), ld[0, 0]


# speedup vs baseline: 14.3354x; 1.9127x over previous
import jax
import jax.numpy as jnp
from jax.experimental import pallas as pl
from jax.experimental.pallas import tpu as pltpu

_CH = 80_000
_NT = 16


def _flow_kernel(b_ref, z_ref, u_ref, w_ref, zo_ref, ld_ref,
                 a_s, c_s, h_s):
    k = pl.program_id(0)
    nk = pl.num_programs(0) // 2

    @pl.when(k == 0)
    def _():
        a_s[...] = jnp.zeros_like(a_s)
        c_s[...] = jnp.zeros_like(c_s)

    @pl.when(k < nk)
    def _():
        w_blk = w_ref[...]
        u_blk = u_ref[...]
        z_blk = z_ref[pl.ds(k * _CH, _CH)]
        a_s[...] += jax.lax.dot_general(
            w_blk, u_blk, (((1,), (1,)), ((), ())),
            preferred_element_type=jnp.float32)
        c_s[...] += jnp.sum(w_blk * z_blk[None, :], axis=1, keepdims=True)

    @pl.when(k == nk)
    def _():
        A = a_s[...]
        base = c_s[...] + b_ref[...]
        acc = jnp.zeros((_NT, 1), jnp.float32)
        hrow = jnp.zeros((1, _NT), jnp.float32)
        ld = jnp.zeros((1, 1), jnp.float32)
        lane = jax.lax.broadcasted_iota(jnp.int32, (1, _NT), 1)
        for t in range(_NT):
            inner = base[t:t + 1, :] + acc[t:t + 1, :]
            h = jnp.tanh(inner)
            d = A[t:t + 1, t:t + 1]
            ld = ld + jnp.log(jnp.abs(1.0 + (1.0 - h * h) * d))
            acc = acc + A[:, t:t + 1] * h
            hrow = hrow + jnp.where(lane == t, h, 0.0)
        h_s[...] = hrow
        ld_ref[...] = ld

    @pl.when(k >= nk)
    def _():
        j = k - nk
        base = j * _CH
        hu = jax.lax.dot_general(
            h_s[...], u_ref[...], (((1,), (0,)), ((), ())),
            preferred_element_type=jnp.float32)[0]
        zv = z_ref[pl.ds(base, _CH)] + hu
        zo_ref[pl.ds(base, 38_400)] = zv[:38_400]
        zo_ref[pl.ds(base + 38_400, 41_600)] = zv[38_400:]


def kernel(z, u, w, b):
    dim = z.shape[0]
    nk = dim // _CH

    z_out, ld = pl.pallas_call(
        _flow_kernel,
        grid=(2 * nk,),
        in_specs=[
            pl.BlockSpec((_NT, 1), lambda k: (0, 0)),
            pl.BlockSpec((dim,), lambda k: (0,)),
            pl.BlockSpec((_NT, _CH), lambda k: (0, jax.lax.rem(k, nk))),
            pl.BlockSpec((_NT, _CH), lambda k: (0, jnp.minimum(k, nk - 1))),
        ],
        out_specs=[
            pl.BlockSpec((dim,), lambda k: (0,)),
            pl.BlockSpec((1, 1), lambda k: (0, 0)),
        ],
        out_shape=[
            jax.ShapeDtypeStruct((dim,), jnp.float32),
            jax.ShapeDtypeStruct((1, 1), jnp.float32),
        ],
        scratch_shapes=[
            pltpu.VMEM((_NT, _NT), jnp.float32),
            pltpu.VMEM((_NT, 1), jnp.float32),
            pltpu.VMEM((1, _NT), jnp.float32),
        ],
        compiler_params=pltpu.CompilerParams(
            dimension_semantics=("arbitrary",),
            vmem_limit_bytes=56 * 1024 * 1024),
        name="flow_fused",
    )(b, z, u, w)

    return z_out, ld[0, 0]
